# baseline (speedup 1.0000x reference)
import functools
import math

import jax
import jax.numpy as jnp
from jax import lax
from jax.experimental import pallas as pl
from jax.experimental.pallas import tpu as pltpu
from jax.experimental.pallas import tpu_sc as plsc

D_EMBED = 64
SCALE = math.sqrt(D_EMBED)

NC, NS = 2, 16
NW = NC * NS
XRC = 4
TB = 2000


def _make_t128(V):
    assert V % TB == 0
    nb = V // TB

    def body(tab_hbm, out_ref, buf, sem):
        i = pl.program_id(0)
        pltpu.make_async_copy(
            tab_hbm.at[pl.ds(i * TB, TB)], buf.at[pl.ds(0, TB)], sem).start()
        pltpu.make_async_copy(
            tab_hbm.at[pl.ds(0, TB)], buf.at[pl.ds(0, TB)], sem).wait()
        nxt = jnp.where(i == nb - 1, 0, (i + 1) * TB)
        pltpu.make_async_copy(
            tab_hbm.at[pl.ds(nxt, 8)], buf.at[pl.ds(TB, 8)], sem).start()
        pltpu.make_async_copy(
            tab_hbm.at[pl.ds(0, 8)], buf.at[pl.ds(TB, 8)], sem).wait()
        out_ref[:, 0:D_EMBED] = buf[0:TB, :]
        out_ref[:, D_EMBED:2 * D_EMBED] = buf[1:TB + 1, :]

    return pl.pallas_call(
        body,
        grid=(nb,),
        in_specs=[pl.BlockSpec(memory_space=pl.ANY)],
        out_specs=pl.BlockSpec((TB, 2 * D_EMBED), lambda i: (i, 0)),
        out_shape=jax.ShapeDtypeStruct((V, 2 * D_EMBED), jnp.float32),
        scratch_shapes=[
            pltpu.VMEM((TB + 8, D_EMBED), jnp.float32),
            pltpu.SemaphoreType.DMA,
        ],
    )


def _make_kernel(R, S):
    assert R % (NW * XRC) == 0
    xr_per_w = R // NW
    n_chunks = xr_per_w // XRC
    assert n_chunks % 2 == 0
    mesh = plsc.VectorSubcoreMesh(
        core_axis_name="c", subcore_axis_name="s",
        num_cores=NC, num_subcores=NS)

    @functools.partial(
        pl.kernel,
        out_type=jax.ShapeDtypeStruct((R, S, D_EMBED), jnp.float32),
        mesh=mesh,
        scratch_types=[
            pltpu.VMEM((xr_per_w, S), jnp.int32),
            pltpu.VMEM((XRC, S, 2 * D_EMBED), jnp.float32),
            pltpu.VMEM((XRC, S, 2 * D_EMBED), jnp.float32),
            pltpu.VMEM((XRC, S, D_EMBED), jnp.float32),
            pltpu.VMEM((XRC, S, D_EMBED), jnp.float32),
            pltpu.SemaphoreType.DMA((2,)),
            pltpu.SemaphoreType.DMA((2,)),
        ],
        compiler_params=pltpu.CompilerParams(use_tc_tiling_on_sc=True),
    )
    def k(x_hbm, t128_hbm, out_hbm, idx_v, gb0, gb1, ob0, ob1, gsem, ssem):
        wid = lax.axis_index("s") * NC + lax.axis_index("c")
        pltpu.sync_copy(x_hbm.at[pl.ds(wid * xr_per_w, xr_per_w)], idx_v)

        gbufs = (gb0, gb1)
        obufs = (ob0, ob1)

        def gather_start(g, b):
            for r in range(XRC):
                pltpu.async_copy(
                    t128_hbm.at[idx_v.at[XRC * g + r]],
                    gbufs[b].at[r], gsem.at[b])

        def gather_wait(b):
            for r in range(XRC):
                pltpu.make_async_copy(
                    t128_hbm.at[idx_v.at[0]], gbufs[b].at[r],
                    gsem.at[b]).wait()

        def store_start(g, b):
            pltpu.async_copy(
                obufs[b], out_hbm.at[pl.ds(wid * xr_per_w + XRC * g, XRC)],
                ssem.at[b])

        def store_wait(b):
            pltpu.make_async_copy(
                obufs[b], out_hbm.at[pl.ds(0, XRC)], ssem.at[b]).wait()

        def scale_out(b):
            gb, ob = gbufs[b], obufs[b]

            def body(s, c):
                for r in range(XRC):
                    for p in range(D_EMBED // 16):
                        ob[r, s, pl.ds(p * 16, 16)] = (
                            gb[r, s, pl.ds(p * 16, 16)] * SCALE)
                return c
            lax.fori_loop(0, S, body, 0)

        gather_start(0, 0)

        def pair(ti, c):
            for ph in range(2):
                g = 2 * ti + ph
                b, nb = ph, 1 - ph

                @pl.when(jnp.logical_and(g >= 1, g + 1 < n_chunks))
                def _():
                    store_wait(nb)

                @pl.when(g + 1 < n_chunks)
                def _():
                    gather_start(g + 1, nb)

                gather_wait(b)
                scale_out(b)
                store_start(g, b)
            return c
        lax.fori_loop(0, n_chunks // 2, pair, 0)
        store_wait(0)
        store_wait(1)

    return k


def kernel(x, table):
    R, S = x.shape
    V = table.shape[0]
    t128 = _make_t128(V)(table)
    return _make_kernel(R, S)(x.astype(jnp.int32), t128)


# speedup vs baseline: 1.3371x; 1.3371x over previous
import functools
import math

import jax
import jax.numpy as jnp
from jax import lax
from jax.experimental import pallas as pl
from jax.experimental.pallas import tpu as pltpu
from jax.experimental.pallas import tpu_sc as plsc

D_EMBED = 64
SCALE = math.sqrt(D_EMBED)

NC, NS = 2, 16
NW = NC * NS
CH = 128
CW = CH * D_EMBED


def _make_kernel(B):
    assert B % (NW * CH) == 0
    n_chunks = B // (NW * CH)
    assert n_chunks % 2 == 0
    b_per_w = B // NW
    mesh = plsc.VectorSubcoreMesh(
        core_axis_name="c", subcore_axis_name="s",
        num_cores=NC, num_subcores=NS)

    @functools.partial(
        pl.kernel,
        out_type=jax.ShapeDtypeStruct((B * D_EMBED,), jnp.float32),
        mesh=mesh,
        scratch_types=[
            pltpu.VMEM((b_per_w,), jnp.int32),
            pltpu.VMEM((CH, D_EMBED), jnp.float32),
            pltpu.VMEM((CH, D_EMBED), jnp.float32),
            pltpu.VMEM((CW,), jnp.float32),
            pltpu.VMEM((CW,), jnp.float32),
            pltpu.SemaphoreType.DMA((2,)),
            pltpu.SemaphoreType.DMA((2,)),
        ],
        compiler_params=pltpu.CompilerParams(use_tc_tiling_on_sc=False),
    )
    def k(x_hbm, table_hbm, out_hbm, idx_v, gb0, gb1, ob0, ob1, gsem, ssem):
        wid = lax.axis_index("s") * NC + lax.axis_index("c")
        pltpu.sync_copy(x_hbm.at[pl.ds(wid * b_per_w, b_per_w)], idx_v)
        gbufs = (gb0, gb1)
        obufs = (ob0, ob1)

        def gather_start(g, b):
            pltpu.async_copy(
                table_hbm.at[idx_v.at[pl.ds(g * CH, CH)]], gbufs[b],
                gsem.at[b])

        def gather_wait(b):
            pltpu.make_async_copy(
                table_hbm.at[idx_v.at[pl.ds(0, CH)]], gbufs[b],
                gsem.at[b]).wait()

        def store_start(g, b):
            pltpu.async_copy(
                obufs[b],
                out_hbm.at[pl.ds((wid * n_chunks + g) * CW, CW)], ssem.at[b])

        def store_wait(b):
            pltpu.make_async_copy(
                obufs[b], out_hbm.at[pl.ds(0, CW)], ssem.at[b]).wait()

        def scale_pack(b):
            gb, ob = gbufs[b], obufs[b]

            def body(r, c):
                for i in range(2):
                    for p in range(D_EMBED // 16):
                        ob[pl.ds(r * 2 * D_EMBED + i * D_EMBED + p * 16, 16)] = (
                            gb[2 * r + i, pl.ds(p * 16, 16)] * SCALE)
                return c
            lax.fori_loop(0, CH // 2, body, 0)

        gather_start(0, 0)

        def pair(t, c):
            for ph in range(2):
                g = 2 * t + ph
                b, nb = ph, 1 - ph

                @pl.when(jnp.logical_and(g >= 1, g + 1 < n_chunks))
                def _():
                    store_wait(nb)

                @pl.when(g + 1 < n_chunks)
                def _():
                    gather_start(g + 1, nb)

                gather_wait(b)
                scale_pack(b)
                store_start(g, b)
            return c
        lax.fori_loop(0, n_chunks // 2, pair, 0)
        store_wait(0)
        store_wait(1)

    return k


def kernel(x, table):
    R, S = x.shape
    B = R * S
    x1d = x.reshape(B).astype(jnp.int32)
    out = _make_kernel(B)(x1d, table)
    return out.reshape(R, S, D_EMBED)


# speedup vs baseline: 1.3731x; 1.0270x over previous
import functools
import math

import jax
import jax.numpy as jnp
from jax import lax
from jax.experimental import pallas as pl
from jax.experimental.pallas import tpu as pltpu
from jax.experimental.pallas import tpu_sc as plsc

D_EMBED = 64
SCALE = math.sqrt(D_EMBED)

NC, NS = 2, 16
NW = NC * NS
XRC = 4


def _make_kernel(R, S):
    assert R % (NW * XRC) == 0
    xr_per_w = R // NW
    n_chunks = xr_per_w // XRC
    assert n_chunks % 2 == 0
    cs = XRC * S
    b_per_w = xr_per_w * S
    splits = []
    o = 0
    while o < cs:
        n = min(128, cs - o)
        splits.append((o, n))
        o += n
    mesh = plsc.VectorSubcoreMesh(
        core_axis_name="c", subcore_axis_name="s",
        num_cores=NC, num_subcores=NS)

    @functools.partial(
        pl.kernel,
        out_type=jax.ShapeDtypeStruct((R, S, D_EMBED), jnp.float32),
        mesh=mesh,
        scratch_types=[
            pltpu.VMEM((b_per_w,), jnp.int32),
            pltpu.VMEM((cs, D_EMBED), jnp.float32),
            pltpu.VMEM((cs, D_EMBED), jnp.float32),
            pltpu.VMEM((XRC, S, D_EMBED), jnp.float32),
            pltpu.VMEM((XRC, S, D_EMBED), jnp.float32),
            pltpu.SemaphoreType.DMA((2,)),
            pltpu.SemaphoreType.DMA((2,)),
        ],
        compiler_params=pltpu.CompilerParams(use_tc_tiling_on_sc=False),
    )
    def k(x_hbm, table_hbm, out_hbm, idx_v, gb0, gb1, ob0, ob1, gsem, ssem):
        wid = lax.axis_index("s") * NC + lax.axis_index("c")
        pltpu.sync_copy(x_hbm.at[pl.ds(wid * b_per_w, b_per_w)], idx_v)
        gbufs = (gb0, gb1)
        obufs = (ob0, ob1)

        def gather_start(g, b):
            for (o, n) in splits:
                pltpu.async_copy(
                    table_hbm.at[idx_v.at[pl.ds(g * cs + o, n)]],
                    gbufs[b].at[pl.ds(o, n)], gsem.at[b])

        def gather_wait(b):
            for (o, n) in splits:
                pltpu.make_async_copy(
                    table_hbm.at[idx_v.at[pl.ds(o, n)]],
                    gbufs[b].at[pl.ds(o, n)], gsem.at[b]).wait()

        def store_start(g, b):
            pltpu.async_copy(
                obufs[b], out_hbm.at[pl.ds(wid * xr_per_w + XRC * g, XRC)],
                ssem.at[b])

        def store_wait(b):
            pltpu.make_async_copy(
                obufs[b], out_hbm.at[pl.ds(0, XRC)], ssem.at[b]).wait()

        def scale_out(b):
            gb, ob = gbufs[b], obufs[b]

            def body(s, c):
                for r in range(XRC):
                    li = r * S + s
                    for p in range(D_EMBED // 16):
                        ob[r, s, pl.ds(p * 16, 16)] = (
                            gb[li, pl.ds(p * 16, 16)] * SCALE)
                return c
            lax.fori_loop(0, S, body, 0)

        gather_start(0, 0)

        def pair(ti, c):
            for ph in range(2):
                g = 2 * ti + ph
                b, nb = ph, 1 - ph

                @pl.when(jnp.logical_and(g >= 1, g + 1 < n_chunks))
                def _():
                    store_wait(nb)

                @pl.when(g + 1 < n_chunks)
                def _():
                    gather_start(g + 1, nb)

                gather_wait(b)
                scale_out(b)
                store_start(g, b)
            return c
        lax.fori_loop(0, n_chunks // 2, pair, 0)
        store_wait(0)
        store_wait(1)

    return k


def kernel(x, table):
    R, S = x.shape
    return _make_kernel(R, S)(x.reshape(R * S).astype(jnp.int32), table)
